# baseline (speedup 1.0000x reference)
import functools

import jax
import jax.numpy as jnp
from jax import lax
from jax.experimental import pallas as pl
from jax.experimental.pallas import tpu as pltpu
from jax.experimental.pallas import tpu_sc as plsc

HOURS = 168
BATCH = 16384

_info = plsc.get_sparse_core_info()
_NC, _NS, _L = _info.num_cores, _info.num_subcores, _info.num_lanes
_NW = _NC * _NS
_BPW = BATCH // _NW
_STEPS = _BPW // _L

_mesh = plsc.VectorSubcoreMesh(core_axis_name="c", subcore_axis_name="s")


@functools.partial(
    pl.kernel,
    mesh=_mesh,
    compiler_params=pltpu.CompilerParams(
        needs_layout_passes=False,
        disable_bounds_checks=True,
        disable_semaphore_checks=True,
        skip_device_barrier=True,
    ),
    out_type=jax.ShapeDtypeStruct((BATCH,), jnp.float32),
    scratch_types=[
        pltpu.VMEM((HOURS,), jnp.float32),
        pltpu.VMEM((HOURS,), jnp.float32),
        pltpu.VMEM((HOURS,), jnp.float32),
        pltpu.VMEM((_BPW,), jnp.int32),
        pltpu.VMEM((_BPW,), jnp.float32),
        pltpu.VMEM((_BPW,), jnp.float32),
        pltpu.VMEM((_BPW,), jnp.float32),
        pltpu.SemaphoreType.DMA,
    ],
)
def _sc_kernel(hour_hbm, x1_hbm, x2_hbm, b0_hbm, b1_hbm, b2_hbm, out_hbm,
               b0_v, b1_v, b2_v, idx_v, x1_v, x2_v, out_v, sem):
    wid = lax.axis_index("s") * _NC + lax.axis_index("c")
    base = wid * _BPW
    cps = [
        pltpu.async_copy(b0_hbm, b0_v, sem),
        pltpu.async_copy(b1_hbm, b1_v, sem),
        pltpu.async_copy(b2_hbm, b2_v, sem),
        pltpu.async_copy(hour_hbm.at[pl.ds(base, _BPW)], idx_v, sem),
        pltpu.async_copy(x1_hbm.at[pl.ds(base, _BPW)], x1_v, sem),
        pltpu.async_copy(x2_hbm.at[pl.ds(base, _BPW)], x2_v, sem),
    ]
    for cp in cps:
        cp.wait()
    for i in range(_STEPS):
        sl = pl.ds(i * _L, _L)
        idx = idx_v[sl]
        g0 = plsc.load_gather(b0_v, [idx])
        g1 = plsc.load_gather(b1_v, [idx])
        g2 = plsc.load_gather(b2_v, [idx])
        out_v[sl] = g0 + g1 * x1_v[sl] + g2 * x2_v[sl]
    pltpu.sync_copy(out_v, out_hbm.at[pl.ds(base, _BPW)])


def kernel(hour_idx, x1, x2, b0, b1, b2):
    return _sc_kernel(hour_idx.astype(jnp.int32), x1, x2, b0, b1, b2)


# speedup vs baseline: 1.0254x; 1.0254x over previous
import functools

import jax
import jax.numpy as jnp
from jax import lax
from jax.experimental import pallas as pl
from jax.experimental.pallas import tpu as pltpu
from jax.experimental.pallas import tpu_sc as plsc

HOURS = 168
BATCH = 16384

_info = plsc.get_sparse_core_info()
_NC, _NS, _L = _info.num_cores, _info.num_subcores, _info.num_lanes
_NW = _NC * _NS
_BPW = BATCH // _NW
_STEPS = _BPW // _L

_mesh = plsc.VectorSubcoreMesh(core_axis_name="c", subcore_axis_name="s")


@functools.partial(
    pl.kernel,
    mesh=_mesh,
    compiler_params=pltpu.CompilerParams(
        needs_layout_passes=False,
        disable_bounds_checks=True,
        disable_semaphore_checks=True,
        skip_device_barrier=True,
    ),
    out_type=jax.ShapeDtypeStruct((BATCH,), jnp.float32),
    scratch_types=[
        pltpu.VMEM((HOURS,), jnp.float32),
        pltpu.VMEM((HOURS,), jnp.float32),
        pltpu.VMEM((HOURS,), jnp.float32),
        pltpu.VMEM((_BPW,), jnp.int32),
        pltpu.VMEM((_BPW,), jnp.float32),
        pltpu.VMEM((_BPW,), jnp.float32),
        pltpu.VMEM((_BPW,), jnp.float32),
        pltpu.SemaphoreType.DMA,
    ],
)
def _sc_kernel(hour_hbm, x1_hbm, x2_hbm, b0_hbm, b1_hbm, b2_hbm, out_hbm,
               b0_v, b1_v, b2_v, idx_v, x1_v, x2_v, out_v, sem):
    wid = lax.axis_index("s") * _NC + lax.axis_index("c")
    base = wid * _BPW
    cps = [
        pltpu.async_copy(b0_hbm, b0_v, sem),
        pltpu.async_copy(b1_hbm, b1_v, sem),
        pltpu.async_copy(b2_hbm, b2_v, sem),
        pltpu.async_copy(hour_hbm.at[pl.ds(base, _BPW)], idx_v, sem),
        pltpu.async_copy(x1_hbm.at[pl.ds(base, _BPW)], x1_v, sem),
        pltpu.async_copy(x2_hbm.at[pl.ds(base, _BPW)], x2_v, sem),
    ]
    for cp in cps:
        cp.wait()
    def body(i, _):
        sl = pl.ds(i * _L, _L)
        idx = idx_v[sl]
        g0 = plsc.load_gather(b0_v, [idx])
        g1 = plsc.load_gather(b1_v, [idx])
        g2 = plsc.load_gather(b2_v, [idx])
        out_v[sl] = g0 + g1 * x1_v[sl] + g2 * x2_v[sl]
        return 0

    lax.fori_loop(0, _STEPS, body, 0, unroll=4)
    pltpu.sync_copy(out_v, out_hbm.at[pl.ds(base, _BPW)])


def kernel(hour_idx, x1, x2, b0, b1, b2):
    return _sc_kernel(hour_idx.astype(jnp.int32), x1, x2, b0, b1, b2)
